# baseline (speedup 1.0000x reference)
import functools
import math

import jax
import jax.numpy as jnp
from jax import lax
from jax.experimental import pallas as pl
from jax.experimental.pallas import tpu as pltpu
from jax.experimental.pallas import tpu_sc as plsc

D_MODEL = 64
SCALE = math.sqrt(D_MODEL)
LANES = 16
BLK = 128
NBUF = 4
LEAD = 2
SKEW = 133
NOB = 2


@functools.partial(jax.jit, static_argnums=(2, 3))
def _embed(x_t, table_p, seq, batch):
    info = plsc.get_sparse_core_info()
    nc, ns = info.num_cores, info.num_subcores
    nw = nc * ns
    assert batch == BLK * nw
    n_chunks = seq
    mesh = plsc.VectorSubcoreMesh(core_axis_name="c", subcore_axis_name="s")

    @functools.partial(
        pl.kernel,
        mesh=mesh,
        out_type=jax.ShapeDtypeStruct((seq, D_MODEL, batch), jnp.float32),
        scratch_types=[
            pltpu.VMEM((seq, BLK), jnp.int32),
            pltpu.VMEM((NBUF, BLK, 128), jnp.float32),
            pltpu.VMEM((D_MODEL, SKEW), jnp.float32),
            pltpu.VMEM((NOB, D_MODEL, BLK), jnp.float32),
            pltpu.SemaphoreType.DMA((NBUF,)),
            pltpu.SemaphoreType.DMA((NOB,)),
        ],
        compiler_params=pltpu.CompilerParams(
            use_tc_tiling_on_sc=True, needs_layout_passes=False
        ),
    )
    def k(
        xt_hbm, table_hbm, out_hbm, strip_v, rows_v, trans_v, outs_v,
        gsem, ssem,
    ):
        wid = lax.axis_index("s") * nc + lax.axis_index("c")
        b0 = wid * BLK
        iota16 = lax.iota(jnp.int32, LANES)
        pltpu.sync_copy(xt_hbm.at[:, pl.ds(b0, BLK)], strip_v)

        def prep_gather(sf, bf):
            pltpu.async_copy(
                table_hbm.at[strip_v.at[sf]], rows_v.at[bf], gsem.at[bf]
            )

        def gather_wait(b):
            pltpu.make_async_copy(
                table_hbm.at[strip_v.at[0]], rows_v.at[b], gsem.at[b]
            ).wait()

        def store_wait(ob):
            pltpu.make_async_copy(
                outs_v.at[ob], out_hbm.at[0, :, pl.ds(0, BLK)], ssem.at[ob]
            ).wait()

        for s in range(LEAD):
            prep_gather(s, s % NBUF)

        n_groups = n_chunks // NBUF

        def group(g, carry):
            for b in range(NBUF):
                s = g * NBUF + b
                sf = s + LEAD
                bf = (b + LEAD) % NBUF

                @pl.when(sf < n_chunks)
                def _():
                    prep_gather(sf, bf)

                gather_wait(b)
                obs = b % NOB

                @pl.when(s >= NOB)
                def _():
                    store_wait(obs)

                dvecs = [iota16 + (dg * LANES) for dg in range(D_MODEL // LANES)]

                @plsc.parallel_loop(0, BLK, step=1, unroll=4)
                def _(j):
                    jb = lax.broadcast(j, (LANES,))
                    for dg in range(D_MODEL // LANES):
                        v = rows_v[b, j, pl.ds(dg * LANES, LANES)]
                        plsc.store_scatter(trans_v, [dvecs[dg], jb], v * SCALE)

                @plsc.parallel_loop(0, D_MODEL, step=1, unroll=4)
                def _(d):
                    for jg in range(BLK // LANES):
                        sl = pl.ds(jg * LANES, LANES)
                        outs_v[obs, d, sl] = trans_v[d, sl]

                pltpu.async_copy(
                    outs_v.at[obs],
                    out_hbm.at[s, :, pl.ds(b0, BLK)],
                    ssem.at[obs],
                )
            return carry

        lax.fori_loop(0, n_groups, group, 0)
        for ob in range(NOB):
            store_wait(ob)

    return k(x_t, table_p)


def kernel(x, embedding_weight):
    b, s = x.shape
    x_t = jnp.swapaxes(x.astype(jnp.int32), 0, 1)
    table_d = jnp.concatenate([embedding_weight, embedding_weight], axis=1)
    out_p = _embed(x_t, table_d, s, b)
    return jnp.transpose(out_p, (2, 0, 1))


# speedup vs baseline: 1.1831x; 1.1831x over previous
import functools
import math

import jax
import jax.numpy as jnp
from jax import lax
from jax.experimental import pallas as pl
from jax.experimental.pallas import tpu as pltpu
from jax.experimental.pallas import tpu_sc as plsc

D_MODEL = 64
SCALE = math.sqrt(D_MODEL)
LANES = 16
CHUNK = 128
NBUF = 8
LEAD = 4


@functools.partial(jax.jit, static_argnums=(2,))
def _embed(x_flat, table, b_flat):
    info = plsc.get_sparse_core_info()
    nc, ns = info.num_cores, info.num_subcores
    nw = nc * ns
    b_per_w = b_flat // nw
    n_chunks = b_per_w // CHUNK
    n_groups = n_chunks // NBUF
    mesh = plsc.VectorSubcoreMesh(core_axis_name="c", subcore_axis_name="s")

    @functools.partial(
        pl.kernel,
        mesh=mesh,
        out_type=jax.ShapeDtypeStruct((b_flat, D_MODEL), jnp.float32),
        scratch_types=[
            pltpu.VMEM((b_per_w,), jnp.int32),
            pltpu.VMEM((NBUF, CHUNK, D_MODEL), jnp.float32),
            pltpu.SemaphoreType.DMA((NBUF,)),
        ],
        compiler_params=pltpu.CompilerParams(use_tc_tiling_on_sc=False),
    )
    def k(x_hbm, table_hbm, out_hbm, idx_v, rows_v, sems):
        wid = lax.axis_index("s") * nc + lax.axis_index("c")
        base = wid * b_per_w
        pltpu.sync_copy(x_hbm.at[pl.ds(base, b_per_w)], idx_v)

        def gather_start(j, b):
            pltpu.async_copy(
                table_hbm.at[idx_v.at[pl.ds(j * CHUNK, CHUNK)]],
                rows_v.at[b],
                sems.at[b],
            )

        def gather_wait(b):
            pltpu.make_async_copy(
                table_hbm.at[idx_v.at[pl.ds(0, CHUNK)]], rows_v.at[b], sems.at[b]
            ).wait()

        def store_start(j, b):
            pltpu.async_copy(
                rows_v.at[b], out_hbm.at[pl.ds(base + j * CHUNK, CHUNK)], sems.at[b]
            )

        def store_wait(b):
            pltpu.make_async_copy(
                rows_v.at[b], out_hbm.at[pl.ds(0, CHUNK)], sems.at[b]
            ).wait()

        for b in range(LEAD):
            gather_start(b, b)

        def group(g, carry):
            for b in range(NBUF):
                j = g * NBUF + b
                bl = (b + LEAD) % NBUF
                jl = j + LEAD

                @pl.when(jl < n_chunks)
                def _():
                    @pl.when(jl >= NBUF)
                    def _():
                        store_wait(bl)

                    gather_start(jl, bl)

                gather_wait(b)

                @plsc.parallel_loop(0, CHUNK, step=1, unroll=4)
                def _(i):
                    for k2 in range(D_MODEL // LANES):
                        sl = (b, i, pl.ds(k2 * LANES, LANES))
                        rows_v[sl] = rows_v[sl] * SCALE

                store_start(j, b)
            return carry

        lax.fori_loop(0, n_groups, group, 0)
        for b in range(NBUF):
            store_wait(b)

    return k(x_flat, table)


def kernel(x, embedding_weight):
    b, s = x.shape
    x_flat = x.reshape(b * s).astype(jnp.int32)
    out = _embed(x_flat, embedding_weight, b * s)
    return out.reshape(b, s, D_MODEL)
